# baseline (speedup 1.0000x reference)
import functools

import jax
import jax.numpy as jnp
from jax import lax
from jax.experimental import pallas as pl
from jax.experimental.pallas import tpu as pltpu
from jax.experimental.pallas import tpu_sc as plsc

E = 100000
H = 256
NC = 2
NS = 16
NW = NC * NS
RB = 40
NCHUNK = E // RB
L = 16
NSLOT = 4
JMAX = -(-NCHUNK // NW) // NSLOT * NSLOT + NSLOT

_mesh = plsc.VectorSubcoreMesh(core_axis_name="c", subcore_axis_name="s")


@functools.partial(
    pl.kernel,
    mesh=_mesh,
    out_type=[
        jax.ShapeDtypeStruct((E, H), jnp.float32),
        jax.ShapeDtypeStruct((E, H), jnp.float32),
    ],
    scratch_types=[
        pltpu.VMEM((NSLOT, RB, H), jnp.float32),
        pltpu.VMEM((NSLOT, RB, H), jnp.float32),
        pltpu.VMEM((NSLOT, RB, H), jnp.float32),
        pltpu.SemaphoreType.DMA((NSLOT,)),
        pltpu.SemaphoreType.DMA((NSLOT,)),
    ],
    compiler_params=pltpu.CompilerParams(use_tc_tiling_on_sc=True),
)
def _sc_gate(x_hbm, y_hbm, t_hbm, o_hbm, g_hbm, tv, xv, yv, sem_in, sem_out):
    wid = lax.axis_index("s") * NC + lax.axis_index("c")
    n_w = (NCHUNK - wid + NW - 1) // NW

    def rows(hbm, j):
        return hbm.at[pl.ds((wid + j * NW) * RB, RB)]

    def in_copies(j, b):
        return (
            pltpu.make_async_copy(rows(t_hbm, j), tv.at[b], sem_in.at[b]),
            pltpu.make_async_copy(rows(x_hbm, j), xv.at[b], sem_in.at[b]),
            pltpu.make_async_copy(rows(y_hbm, j), yv.at[b], sem_in.at[b]),
        )

    def out_copies(j, b):
        return (
            pltpu.make_async_copy(tv.at[b], rows(g_hbm, j), sem_out.at[b]),
            pltpu.make_async_copy(xv.at[b], rows(o_hbm, j), sem_out.at[b]),
        )

    def start_in(j, b):
        @pl.when(j < n_w)
        def _():
            for c in in_copies(j, b):
                c.start()

    def wait_in(j, b):
        @pl.when(j < n_w)
        def _():
            for c in in_copies(j, b):
                c.wait()

    def start_out(j, b):
        @pl.when(j < n_w)
        def _():
            for c in out_copies(j, b):
                c.start()

    def wait_out(j, b):
        @pl.when(jnp.logical_and(j >= 0, j < n_w))
        def _():
            for c in out_copies(j, b):
                c.wait()

    def compute(j, b):
        @pl.when(j < n_w)
        def _():
            def row_body(r, carry):
                for c in range(H // L):
                    s = pl.ds(c * L, L)
                    t = tv[b, r, s]
                    g = 1.0 / (1.0 + jnp.exp(-t))
                    tv[b, r, s] = g
                    y = yv[b, r, s]
                    xv[b, r, s] = y + g * (xv[b, r, s] - y)
                return carry

            lax.fori_loop(0, RB, row_body, 0)

    for b in range(NSLOT):
        start_in(b, b)

    def step(i, carry):
        j = i * NSLOT
        for b in range(NSLOT):
            jj = j + b
            wait_in(jj, b)
            compute(jj, b)
            start_out(jj, b)
            wait_out(jj - NSLOT, b)
            start_in(jj + NSLOT, b)
        return carry

    lax.fori_loop(0, JMAX // NSLOT, step, 0)
    for b in range(NSLOT):
        wait_out(JMAX - NSLOT + b, b)


def kernel(X, Y, gate_theta):
    o, g = _sc_gate(X, Y, gate_theta)
    return (o, g)
